# baseline (speedup 1.0000x reference)
import functools

import jax
import jax.numpy as jnp
import numpy as np
from jax import lax
from jax.experimental import pallas as pl
from jax.experimental.pallas import tpu as pltpu
from jax.experimental.pallas import tpu_sc as plsc

B = 1024
C = 26
S = 50
VOCAB = 100000
EMB = 32

N = B * C * S
NW = 32
PER_W = N // NW
BLK = 128
NBLK = PER_W // BLK
LANES = 16

_mesh = plsc.VectorSubcoreMesh(core_axis_name="c", subcore_axis_name="s")


@functools.partial(
    pl.kernel,
    mesh=_mesh,
    out_type=jax.ShapeDtypeStruct((C, S, EMB, B), jnp.float32),
    scratch_types=[
        pltpu.VMEM((PER_W,), jnp.int32),
        pltpu.VMEM((PER_W,), jnp.int32),
        pltpu.VMEM((BLK, EMB), jnp.float32),
        pltpu.VMEM((BLK, EMB), jnp.float32),
        pltpu.VMEM((EMB, BLK), jnp.float32),
        pltpu.VMEM((EMB, BLK), jnp.float32),
        pltpu.SemaphoreType.DMA,
        pltpu.SemaphoreType.DMA,
        pltpu.SemaphoreType.DMA,
        pltpu.SemaphoreType.DMA,
        pltpu.SemaphoreType.DMA,
    ],
    compiler_params=pltpu.CompilerParams(
        use_tc_tiling_on_sc=False, needs_layout_passes=False
    ),
)
def _gather_kernel(x_hbm, tab_hbm, offs_hbm, out_hbm, idx_v, offs_v,
                   rows_a, rows_b, blk_a, blk_b, in_sem, ga_sem, gb_sem,
                   oa_sem, ob_sem):
    wid = lax.axis_index("s") * 2 + lax.axis_index("c")
    base = wid * PER_W
    jbase = wid * NBLK

    pltpu.async_copy(x_hbm.at[pl.ds(base, PER_W)], idx_v, in_sem)
    pltpu.async_copy(offs_hbm.at[pl.ds(base, PER_W)], offs_v, in_sem)
    pltpu.make_async_copy(x_hbm.at[pl.ds(base, PER_W)], idx_v, in_sem).wait()
    pltpu.make_async_copy(offs_hbm.at[pl.ds(base, PER_W)], offs_v, in_sem).wait()

    def add_body(j, c):
        sl = pl.ds(j * LANES, LANES)
        idx_v[sl] = idx_v[sl] + offs_v[sl]
        return c
    lax.fori_loop(0, PER_W // LANES, add_body, 0)

    def start_gather(k, rows, sem):
        pltpu.async_copy(tab_hbm.at[idx_v.at[pl.ds(k * BLK, BLK)]], rows, sem)

    def wait_gather(rows, sem):
        pltpu.make_async_copy(tab_hbm.at[idx_v.at[pl.ds(0, BLK)]], rows, sem).wait()

    iota16 = lax.broadcasted_iota(jnp.int32, (LANES,), 0)
    zero16 = iota16 * 0

    def transform(rows, blk):
        for e in range(EMB):
            col = zero16 + e
            for g in range(8):
                row = iota16 + (g * LANES)
                blk[e, pl.ds(g * LANES, LANES)] = plsc.load_gather(rows, [row, col])

    def out_ref(k):
        j = jbase + k
        p = jax.lax.shift_right_logical(j, 3)
        b0 = (j & 7) * BLK
        cc = p // S
        ss = p % S
        return out_hbm.at[cc, ss, :, pl.ds(b0, BLK)]

    def start_out(k, blk, sem):
        pltpu.async_copy(blk, out_ref(k), sem)

    def wait_out(k, blk, sem):
        pltpu.make_async_copy(blk, out_ref(k), sem).wait()

    start_gather(0, rows_a, ga_sem)

    @pl.loop(0, NBLK // 2)
    def _pipe(i):
        k0 = 2 * i
        k1 = 2 * i + 1
        start_gather(k1, rows_b, gb_sem)
        wait_gather(rows_a, ga_sem)

        @pl.when(i >= 1)
        def _():
            wait_out(k0 - 2, blk_a, oa_sem)

        transform(rows_a, blk_a)
        start_out(k0, blk_a, oa_sem)
        start_gather(k0 + 2, rows_a, ga_sem)
        wait_gather(rows_b, gb_sem)

        @pl.when(i >= 1)
        def _():
            wait_out(k1 - 2, blk_b, ob_sem)

        transform(rows_b, blk_b)
        start_out(k1, blk_b, ob_sem)

    last = NBLK - 1
    wait_gather(rows_a, ga_sem)
    wait_out(last - 2, blk_a, oa_sem)
    transform(rows_a, blk_a)
    start_out(last, blk_a, oa_sem)
    wait_out(last - 1, blk_b, ob_sem)
    wait_out(last, blk_a, oa_sem)


_OFFS_T = np.repeat(np.arange(C, dtype=np.int32) * VOCAB, S * B)


def kernel(X, tables):
    x_t = jnp.transpose(X, (1, 2, 0)).reshape(N)
    tab_flat = tables.reshape(C * VOCAB, EMB)
    y = _gather_kernel(x_t, tab_flat, jnp.asarray(_OFFS_T))
    return jnp.transpose(y, (3, 0, 1, 2))
